# baseline (speedup 1.0000x reference)
import functools
import math

import jax
import jax.numpy as jnp
from jax import lax
from jax.experimental import pallas as pl
from jax.experimental.pallas import tpu as pltpu
from jax.experimental.pallas import tpu_sc as plsc

N_ROWS = 2048
SIZE = 32000
PAD = 0
EPS = 0.1 / (SIZE - 2)
CONF = 0.9
C0 = 0.1 * math.log(EPS) + CONF * math.log(CONF)

NC = 2
NS = 16
NW = NC * NS
LANES = 16

ROWS_TC = 1280
ROWS_SC = N_ROWS - ROWS_TC
NR = ROWS_SC // NW

CROWS = 8
NCH = ROWS_TC // CROWS
NBUF = 20

_ROW_UNROLL = 16
_ROW_ITERS = SIZE // (LANES * _ROW_UNROLL)


def _sc_row_sum(buf):
    def chunk(i, a):
        base = i * (LANES * _ROW_UNROLL)
        for k in range(_ROW_UNROLL):
            a = a + buf[pl.ds(base + k * LANES, LANES)]
        return a

    return lax.fori_loop(0, _ROW_ITERS, chunk, jnp.zeros((LANES,), jnp.float32))


def _sc_body(x_hbm, out_hbm, rowbuf, parts_v, rsem0, rsem1):
    wid = lax.axis_index("s") * NC + lax.axis_index("c")
    rbase = ROWS_TC + wid * NR
    rsems = (rsem0, rsem1)

    def row_copy(r, slot):
        return pltpu.make_async_copy(
            x_hbm.at[rbase + r], rowbuf.at[slot], rsems[slot]
        )

    row_copy(0, 0).start()
    row_copy(1, 1).start()
    for r in range(NR):
        slot = r % 2
        row_copy(r, slot).wait()
        if r + 2 < NR:
            row_copy(r + 2, slot).start()
        parts_v[r] = _sc_row_sum(rowbuf.at[slot])
    pltpu.sync_copy(parts_v, out_hbm.at[pl.ds(wid * NR, NR)])


@functools.cache
def _sc_kernel():
    return pl.kernel(
        _sc_body,
        mesh=plsc.VectorSubcoreMesh(core_axis_name="c", subcore_axis_name="s"),
        out_type=jax.ShapeDtypeStruct((ROWS_SC, LANES), jnp.float32),
        scratch_types=[
            pltpu.VMEM((2, SIZE), jnp.float32),
            pltpu.VMEM((NR, LANES), jnp.float32),
            pltpu.SemaphoreType.DMA,
            pltpu.SemaphoreType.DMA,
        ],
        cost_estimate=pl.CostEstimate(
            flops=ROWS_SC * SIZE,
            transcendentals=0,
            bytes_accessed=ROWS_SC * SIZE * 4,
        ),
    )


def _tc_body(x_hbm, ts_ref, tv_ref, out_ref, bufs, gbuf, x0buf, sems, gsem):
    def issue(c, k):
        pltpu.make_async_copy(
            x_hbm.at[pl.ds(c * CROWS, CROWS), :], bufs.at[k], sems.at[k]
        ).start()

    for k in range(NBUF):
        issue(k, k)

    def g_copy(i):
        c128 = (ts_ref[i] // 128) * 128
        return pltpu.make_async_copy(
            x_hbm.at[pl.ds(i, 1), pl.ds(c128, 128)],
            gbuf.at[pl.ds(i - ROWS_TC, 1), :], gsem,
        )

    def x0_copy(i):
        return pltpu.make_async_copy(
            x_hbm.at[pl.ds(i, 1), pl.ds(0, 128)],
            x0buf.at[pl.ds(i - ROWS_TC, 1), :], gsem,
        )

    def gissue(i, carry):
        g_copy(i).start()
        x0_copy(i).start()
        return carry

    n_outer = NCH // NBUF
    gpi = -(-ROWS_SC // n_outer)

    def outer(o, carry):
        acc, x0acc, gacc, macc = carry
        lax.fori_loop(
            ROWS_TC + o * gpi,
            jnp.minimum(ROWS_TC + (o + 1) * gpi, N_ROWS),
            gissue,
            0,
        )
        for k in range(NBUF):
            c = o * NBUF + k
            pltpu.make_async_copy(
                x_hbm.at[pl.ds(c * CROWS, CROWS), :], bufs.at[k], sems.at[k]
            ).wait()
            chunk = bufs[k]
            tc16 = tv_ref[pl.ds(c * CROWS, CROWS), :]
            m = (tc16 != PAD).astype(jnp.float32)
            rs = jnp.sum(chunk, axis=1, keepdims=True)
            sel = tc16 == lax.broadcasted_iota(jnp.int32, (CROWS, SIZE), 1)
            gs = jnp.sum(jnp.where(sel, chunk, 0.0), axis=1, keepdims=True)
            acc = acc + rs * m
            x0acc = x0acc + chunk[:, 0:1] * m
            gacc = gacc + gs * m
            macc = macc + m
            nc = c + NBUF

            @pl.when(nc < NCH)
            def _prefetch():
                issue(nc, k)

        return acc, x0acc, gacc, macc

    zero81 = jnp.zeros((CROWS, 1), jnp.float32)
    acc, x0acc, gacc, macc = lax.fori_loop(
        0, n_outer, outer, (zero81, zero81, zero81, zero81)
    )

    def gwait(i, carry):
        g_copy(i).wait()
        x0_copy(i).wait()
        return carry

    lax.fori_loop(ROWS_TC, N_ROWS, gwait, 0)

    t_sc = tv_ref[ROWS_TC:, :]
    lane = t_sc % 128
    sel = lane == lax.broadcasted_iota(jnp.int32, (1, 128), 1)
    g_sc = jnp.sum(jnp.where(sel, gbuf[...], 0.0), axis=1, keepdims=True)
    sparse_sc = jnp.where(
        t_sc != PAD, (EPS - CONF) * g_sc + EPS * x0buf[:, 0:1] + C0, 0.0
    )
    out_ref[0, 0] = (
        jnp.sum(sparse_sc)
        + (EPS - CONF) * jnp.sum(gacc)
        + EPS * jnp.sum(x0acc)
        + C0 * jnp.sum(macc)
        - EPS * jnp.sum(acc)
    )


def _combine_body(scb_ref, tv_ref, tc_ref, out_ref):
    rows = jnp.sum(scb_ref[...], axis=1, keepdims=True)
    m = (tv_ref[ROWS_TC:, :] != PAD).astype(jnp.float32)
    out_ref[0, 0] = tc_ref[0, 0] - EPS * jnp.sum(rows * m)


def kernel(x, target):
    t_col = target.reshape(N_ROWS, 1)
    sc_bulk = _sc_kernel()(x)
    tc_part = pl.pallas_call(
        _tc_body,
        in_specs=[
            pl.BlockSpec(memory_space=pl.ANY),
            pl.BlockSpec(memory_space=pltpu.SMEM),
            pl.BlockSpec(memory_space=pltpu.VMEM),
        ],
        out_specs=pl.BlockSpec(memory_space=pltpu.SMEM),
        out_shape=jax.ShapeDtypeStruct((1, 1), jnp.float32),
        scratch_shapes=[
            pltpu.VMEM((NBUF, CROWS, SIZE), jnp.float32),
            pltpu.VMEM((ROWS_SC, 128), jnp.float32),
            pltpu.VMEM((ROWS_SC, 128), jnp.float32),
            pltpu.SemaphoreType.DMA((NBUF,)),
            pltpu.SemaphoreType.DMA,
        ],
    )(x, target, t_col)
    out = pl.pallas_call(
        _combine_body,
        in_specs=[
            pl.BlockSpec(memory_space=pltpu.VMEM),
            pl.BlockSpec(memory_space=pltpu.VMEM),
            pl.BlockSpec(memory_space=pltpu.SMEM),
        ],
        out_specs=pl.BlockSpec(memory_space=pltpu.SMEM),
        out_shape=jax.ShapeDtypeStruct((1, 1), jnp.float32),
    )(sc_bulk, t_col, tc_part)
    return out.reshape(())


# speedup vs baseline: 1.0018x; 1.0018x over previous
import functools
import math

import jax
import jax.numpy as jnp
from jax import lax
from jax.experimental import pallas as pl
from jax.experimental.pallas import tpu as pltpu
from jax.experimental.pallas import tpu_sc as plsc

N_ROWS = 2048
SIZE = 32000
PAD = 0
EPS = 0.1 / (SIZE - 2)
CONF = 0.9
C0 = 0.1 * math.log(EPS) + CONF * math.log(CONF)

NC = 2
NS = 16
NW = NC * NS
LANES = 16

ROWS_TC = 1280
ROWS_SC = N_ROWS - ROWS_TC
NR = ROWS_SC // NW

CROWS = 8
NCH = ROWS_TC // CROWS
NBUF = 20

_ROW_UNROLL = 16
_ROW_ITERS = SIZE // (LANES * _ROW_UNROLL)


def _sc_row_sum(buf):
    def chunk(i, a):
        base = i * (LANES * _ROW_UNROLL)
        for k in range(_ROW_UNROLL):
            a = a + buf[pl.ds(base + k * LANES, LANES)]
        return a

    return lax.fori_loop(0, _ROW_ITERS, chunk, jnp.zeros((LANES,), jnp.float32))


_SC_RING = 2


def _sc_body(x_hbm, out_hbm, rowbuf, parts_v, rsem0, rsem1):
    wid = lax.axis_index("s") * NC + lax.axis_index("c")
    rbase = ROWS_TC + wid * NR
    rsems = (rsem0, rsem1)

    def row_copy(r, slot):
        return pltpu.make_async_copy(
            x_hbm.at[rbase + r], rowbuf.at[slot], rsems[slot]
        )

    for s in range(_SC_RING):
        row_copy(s, s).start()
    for r in range(NR):
        slot = r % _SC_RING
        row_copy(r, slot).wait()
        if r + _SC_RING < NR:
            row_copy(r + _SC_RING, slot).start()
        parts_v[r] = _sc_row_sum(rowbuf.at[slot])
    pltpu.sync_copy(parts_v, out_hbm.at[pl.ds(wid * NR, NR)])


@functools.cache
def _sc_kernel():
    return pl.kernel(
        _sc_body,
        mesh=plsc.VectorSubcoreMesh(core_axis_name="c", subcore_axis_name="s"),
        out_type=jax.ShapeDtypeStruct((ROWS_SC, LANES), jnp.float32),
        scratch_types=[
            pltpu.VMEM((2, SIZE), jnp.float32),
            pltpu.VMEM((NR, LANES), jnp.float32),
            pltpu.SemaphoreType.DMA,
            pltpu.SemaphoreType.DMA,
        ],
        cost_estimate=pl.CostEstimate(
            flops=ROWS_SC * SIZE,
            transcendentals=0,
            bytes_accessed=ROWS_SC * SIZE * 4,
        ),
    )


def _tc_body(x_hbm, ts_ref, tv_ref, out_ref, bufs, gbuf, x0buf, sems, gsem):
    def issue(c, k):
        pltpu.make_async_copy(
            x_hbm.at[pl.ds(c * CROWS, CROWS), :], bufs.at[k], sems.at[k]
        ).start()

    for k in range(NBUF):
        issue(k, k)

    def g_copy(i):
        c128 = (ts_ref[i] // 128) * 128
        return pltpu.make_async_copy(
            x_hbm.at[pl.ds(i, 1), pl.ds(c128, 128)],
            gbuf.at[pl.ds(i - ROWS_TC, 1), :], gsem,
        )

    def x0_copy(i):
        return pltpu.make_async_copy(
            x_hbm.at[pl.ds(i, 1), pl.ds(0, 128)],
            x0buf.at[pl.ds(i - ROWS_TC, 1), :], gsem,
        )

    def gissue(i, carry):
        g_copy(i).start()
        x0_copy(i).start()
        return carry

    n_outer = NCH // NBUF
    gpi = -(-ROWS_SC // n_outer)

    def outer(o, carry):
        acc, x0acc, gacc, macc = carry
        lax.fori_loop(
            ROWS_TC + o * gpi,
            jnp.minimum(ROWS_TC + (o + 1) * gpi, N_ROWS),
            gissue,
            0,
        )
        for k in range(NBUF):
            c = o * NBUF + k
            pltpu.make_async_copy(
                x_hbm.at[pl.ds(c * CROWS, CROWS), :], bufs.at[k], sems.at[k]
            ).wait()
            chunk = bufs[k]
            tc16 = tv_ref[pl.ds(c * CROWS, CROWS), :]
            m = (tc16 != PAD).astype(jnp.float32)
            rs = jnp.sum(chunk, axis=1, keepdims=True)
            sel = tc16 == lax.broadcasted_iota(jnp.int32, (CROWS, SIZE), 1)
            gs = jnp.sum(jnp.where(sel, chunk, 0.0), axis=1, keepdims=True)
            acc = acc + rs * m
            x0acc = x0acc + chunk[:, 0:1] * m
            gacc = gacc + gs * m
            macc = macc + m
            nc = c + NBUF

            @pl.when(nc < NCH)
            def _prefetch():
                issue(nc, k)

        return acc, x0acc, gacc, macc

    zero81 = jnp.zeros((CROWS, 1), jnp.float32)
    acc, x0acc, gacc, macc = lax.fori_loop(
        0, n_outer, outer, (zero81, zero81, zero81, zero81)
    )

    def gwait(i, carry):
        g_copy(i).wait()
        x0_copy(i).wait()
        return carry

    lax.fori_loop(ROWS_TC, N_ROWS, gwait, 0)

    t_sc = tv_ref[ROWS_TC:, :]
    lane = t_sc % 128
    sel = lane == lax.broadcasted_iota(jnp.int32, (1, 128), 1)
    g_sc = jnp.sum(jnp.where(sel, gbuf[...], 0.0), axis=1, keepdims=True)
    sparse_sc = jnp.where(
        t_sc != PAD, (EPS - CONF) * g_sc + EPS * x0buf[:, 0:1] + C0, 0.0
    )
    out_ref[0, 0] = (
        jnp.sum(sparse_sc)
        + (EPS - CONF) * jnp.sum(gacc)
        + EPS * jnp.sum(x0acc)
        + C0 * jnp.sum(macc)
        - EPS * jnp.sum(acc)
    )


def _combine_body(scb_ref, tv_ref, tc_ref, out_ref):
    rows = jnp.sum(scb_ref[...], axis=1, keepdims=True)
    m = (tv_ref[ROWS_TC:, :] != PAD).astype(jnp.float32)
    out_ref[0, 0] = tc_ref[0, 0] - EPS * jnp.sum(rows * m)


def kernel(x, target):
    t_col = target.reshape(N_ROWS, 1)
    sc_bulk = _sc_kernel()(x)
    tc_part = pl.pallas_call(
        _tc_body,
        in_specs=[
            pl.BlockSpec(memory_space=pl.ANY),
            pl.BlockSpec(memory_space=pltpu.SMEM),
            pl.BlockSpec(memory_space=pltpu.VMEM),
        ],
        out_specs=pl.BlockSpec(memory_space=pltpu.SMEM),
        out_shape=jax.ShapeDtypeStruct((1, 1), jnp.float32),
        scratch_shapes=[
            pltpu.VMEM((NBUF, CROWS, SIZE), jnp.float32),
            pltpu.VMEM((ROWS_SC, 128), jnp.float32),
            pltpu.VMEM((ROWS_SC, 128), jnp.float32),
            pltpu.SemaphoreType.DMA((NBUF,)),
            pltpu.SemaphoreType.DMA,
        ],
    )(x, target, t_col)
    out = pl.pallas_call(
        _combine_body,
        in_specs=[
            pl.BlockSpec(memory_space=pltpu.VMEM),
            pl.BlockSpec(memory_space=pltpu.VMEM),
            pl.BlockSpec(memory_space=pltpu.SMEM),
        ],
        out_specs=pl.BlockSpec(memory_space=pltpu.SMEM),
        out_shape=jax.ShapeDtypeStruct((1, 1), jnp.float32),
    )(sc_bulk, t_col, tc_part)
    return out.reshape(())
